# baseline (speedup 1.0000x reference)
import jax
import jax.numpy as jnp
from jax.experimental import pallas as pl
from jax.experimental.pallas import tpu as pltpu

CB = 2


def _qsample_body(t_ref, c1tab_ref, c2tab_ref, x_ref, n_ref, o_ref):
    b = pl.program_id(0)
    tt = t_ref[b]
    c1 = c1tab_ref[tt]
    c2 = c2tab_ref[tt]
    o_ref[...] = x_ref[...] * c1 + n_ref[...] * c2


def kernel(x_start, t, noise, sqrt_alphas_cumprod, sqrt_one_minus_alphas_cumprod):
    B, C, H, W = x_start.shape
    grid = (B, C // CB)

    data_spec = pl.BlockSpec((1, CB, H, W), lambda b, c, *_: (b, c, 0, 0))
    return pl.pallas_call(
        _qsample_body,
        grid_spec=pltpu.PrefetchScalarGridSpec(
            num_scalar_prefetch=3,
            grid=grid,
            in_specs=[data_spec, data_spec],
            out_specs=data_spec,
        ),
        out_shape=jax.ShapeDtypeStruct((B, C, H, W), x_start.dtype),
        compiler_params=pltpu.CompilerParams(
            dimension_semantics=("parallel", "arbitrary"),
        ),
    )(t, sqrt_alphas_cumprod, sqrt_one_minus_alphas_cumprod, x_start, noise)


# speedup vs baseline: 1.4361x; 1.4361x over previous
import jax
import jax.numpy as jnp
from jax.experimental import pallas as pl
from jax.experimental.pallas import tpu as pltpu

CB = 4


def _qsample_body(t_ref, c1tab_ref, c2tab_ref, x_ref, n_ref, o_ref):
    b = pl.program_id(0)
    tt = t_ref[b]
    c1 = c1tab_ref[tt]
    c2 = c2tab_ref[tt]
    o_ref[...] = x_ref[...] * c1 + n_ref[...] * c2


def kernel(x_start, t, noise, sqrt_alphas_cumprod, sqrt_one_minus_alphas_cumprod):
    B, C, H, W = x_start.shape
    grid = (B, C // CB)

    data_spec = pl.BlockSpec((1, CB, H, W), lambda b, c, *_: (b, c, 0, 0))
    return pl.pallas_call(
        _qsample_body,
        grid_spec=pltpu.PrefetchScalarGridSpec(
            num_scalar_prefetch=3,
            grid=grid,
            in_specs=[data_spec, data_spec],
            out_specs=data_spec,
        ),
        out_shape=jax.ShapeDtypeStruct((B, C, H, W), x_start.dtype),
        compiler_params=pltpu.CompilerParams(
            dimension_semantics=("parallel", "arbitrary"),
        ),
    )(t, sqrt_alphas_cumprod, sqrt_one_minus_alphas_cumprod, x_start, noise)


# speedup vs baseline: 1.7980x; 1.2520x over previous
import jax
import jax.numpy as jnp
from jax.experimental import pallas as pl
from jax.experimental.pallas import tpu as pltpu

CB = 8


def _qsample_body(t_ref, c1tab_ref, c2tab_ref, x_ref, n_ref, o_ref):
    b = pl.program_id(0)
    tt = t_ref[b]
    c1 = c1tab_ref[tt]
    c2 = c2tab_ref[tt]
    o_ref[...] = x_ref[...] * c1 + n_ref[...] * c2


def kernel(x_start, t, noise, sqrt_alphas_cumprod, sqrt_one_minus_alphas_cumprod):
    B, C, H, W = x_start.shape
    grid = (B, C // CB)

    data_spec = pl.BlockSpec((1, CB, H, W), lambda b, c, *_: (b, c, 0, 0))
    return pl.pallas_call(
        _qsample_body,
        grid_spec=pltpu.PrefetchScalarGridSpec(
            num_scalar_prefetch=3,
            grid=grid,
            in_specs=[data_spec, data_spec],
            out_specs=data_spec,
        ),
        out_shape=jax.ShapeDtypeStruct((B, C, H, W), x_start.dtype),
        compiler_params=pltpu.CompilerParams(
            dimension_semantics=("parallel", "arbitrary"),
        ),
    )(t, sqrt_alphas_cumprod, sqrt_one_minus_alphas_cumprod, x_start, noise)
